# baseline (speedup 1.0000x reference)
import dataclasses
import functools

import jax
import jax.numpy as jnp
from jax import lax
from jax.experimental import pallas as pl
from jax.experimental.pallas import tpu as pltpu
from jax.experimental.pallas import tpu_sc as plsc

N = 50000
D = 256
B = 256
C_MAX = 32
CHUNK = 64
NCHUNKS = (N + CHUNK - 1) // CHUNK
NBUCKET = B * C_MAX
HALF = NBUCKET // 2
NSUB = 16
NWORK = 2 * NSUB
ROWS_PAD = NBUCKET + NWORK
ZROWS = NBUCKET // NWORK
KMAX = -(-NCHUNKS // NSUB)
NPAD = 49 * 1024


def _sc_body(node_hbm, batch_hbm, comp_hbm, zeros_hbm, e_out,
             nlo, nhi, bbufs, cbufs, ibufs, flags, acc_lo, acc_hi,
             semz, semh, semn0, semn1, sems0, sems1):
    cid = lax.axis_index("c")
    sid = lax.axis_index("s")
    base = cid * HALF
    glo = cid * (B // 2)
    semn = (semn0, semn1)
    sems = (sems0, sems1)

    def chunk_start(k):
        j = k * NSUB + sid
        return jnp.minimum(j * CHUNK, N - CHUNK)

    def header_dma(k):
        start = chunk_start(k)
        slot = lax.rem(k, 4) if isinstance(k, jax.Array) else k % 4
        return (pltpu.make_async_copy(batch_hbm.at[pl.ds(start, CHUNK)],
                                      bbufs.at[slot], semh),
                pltpu.make_async_copy(comp_hbm.at[pl.ds(start, CHUNK)],
                                      cbufs.at[slot], semh))

    zl = pltpu.make_async_copy(zeros_hbm, acc_lo.at[pl.ds(sid * ZROWS, ZROWS)],
                               semz)
    zh = pltpu.make_async_copy(zeros_hbm, acc_hi.at[pl.ds(sid * ZROWS, ZROWS)],
                               semz)
    zl.start()
    zh.start()
    for k in range(min(2, KMAX)):
        for h in header_dma(k):
            h.start()

    iota = lax.broadcasted_iota(jnp.int32, (16,), 0)
    dummy = HALF + sid
    HD = D // 2

    zl.wait()
    zh.wait()
    plsc.subcore_barrier()

    def node_dma(k, half):
        buf = lax.rem(k, 2)
        src = node_hbm.at[pl.ds(chunk_start(k), CHUNK), pl.ds(half * HD, HD)]
        return pltpu.make_async_copy(src, (nlo, nhi)[half].at[buf],
                                     semn[half])

    def scat_dma(k, half):
        buf = lax.rem(k, 2)
        return pltpu.make_async_copy(
            (nlo, nhi)[half].at[buf],
            (acc_lo, acc_hi)[half].at[ibufs.at[lax.rem(k, 4)]], sems[half])

    @pl.loop(0, KMAX + 2)
    def _(k):
        @pl.when((k >= 2) & (flags[jnp.maximum(k - 2, 0)] == 1))
        def _():
            scat_dma(k - 2, 0).wait()
            scat_dma(k - 2, 1).wait()

        @pl.when(k + 2 < KMAX)
        def _():
            for h in header_dma(k + 2):
                h.start()

        @pl.when(k < KMAX)
        def _():
            j = k * NSUB + sid
            start = chunk_start(k)
            slot = lax.rem(k, 4)
            for h in header_dma(k):
                h.wait()
            lo = jnp.min(bbufs[slot, pl.ds(0, 16)])
            hi = jnp.max(bbufs[slot, pl.ds(CHUNK - 16, 16)])
            rel = (hi >= glo) & (lo < glo + B // 2) & (j < NCHUNKS)
            flags[k] = rel.astype(jnp.int32)

            @pl.when(rel)
            def _():
                node_dma(k, 0).start()
                node_dma(k, 1).start()
                for g in range(CHUNK // 16):
                    b = bbufs[slot, pl.ds(g * 16, 16)]
                    c = cbufs[slot, pl.ds(g * 16, 16)]
                    sval = b * C_MAX + c - base
                    pos = start + g * 16 + iota
                    ok = (sval >= 0) & (sval < HALF) & (pos >= j * CHUNK)
                    ibufs[slot, pl.ds(g * 16, 16)] = jnp.where(ok, sval, dummy)

        @pl.when((k >= 1) & (k <= KMAX)
                 & (flags[jnp.maximum(k - 1, 0)] == 1))
        def _():
            node_dma(k - 1, 0).wait()
            node_dma(k - 1, 1).wait()
            scat_dma(k - 1, 0).start(add=True)
            scat_dma(k - 1, 1).start(add=True)

    plsc.subcore_barrier()
    row = sid * ZROWS
    pltpu.sync_copy(acc_lo.at[pl.ds(row, ZROWS)],
                    e_out.at[pl.ds(base + row, ZROWS), pl.ds(0, HD)])
    pltpu.sync_copy(acc_hi.at[pl.ds(row, ZROWS)],
                    e_out.at[pl.ds(base + row, ZROWS), pl.ds(HD, HD)])


_sc_compiler_params = pltpu.CompilerParams()
if "needs_layout_passes" in pltpu.CompilerParams.__dataclass_fields__:
    _sc_compiler_params = dataclasses.replace(
        _sc_compiler_params, needs_layout_passes=False)

_sc_segment_sum = functools.partial(
    pl.kernel,
    compiler_params=_sc_compiler_params,
    out_type=jax.ShapeDtypeStruct((NBUCKET, D), jnp.float32),
    mesh=plsc.VectorSubcoreMesh(core_axis_name="c", subcore_axis_name="s"),
    scratch_types=[
        pltpu.VMEM((2, CHUNK, D // 2), jnp.float32),
        pltpu.VMEM((2, CHUNK, D // 2), jnp.float32),
        pltpu.VMEM((4, CHUNK), jnp.int32),
        pltpu.VMEM((4, CHUNK), jnp.int32),
        pltpu.VMEM((4, CHUNK), jnp.int32),
        pltpu.SMEM((KMAX,), jnp.int32),
        pltpu.VMEM_SHARED((HALF + NSUB, D // 2), jnp.float32),
        pltpu.VMEM_SHARED((HALF + NSUB, D // 2), jnp.float32),
        pltpu.SemaphoreType.DMA,
        pltpu.SemaphoreType.DMA,
        pltpu.SemaphoreType.DMA,
        pltpu.SemaphoreType.DMA,
        pltpu.SemaphoreType.DMA,
        pltpu.SemaphoreType.DMA,
    ],
)(_sc_body)


NC_ROWS = 8
CHK = 128


def _numc_body(b_ref, c_ref, o_ref, mx_ref):
    i = pl.program_id(0)

    @pl.when(i == 0)
    def _():
        mx_ref[...] = jnp.full((1, B), -1, jnp.int32)

    giota = lax.broadcasted_iota(jnp.int32, (1, B), 1)
    bt = jnp.transpose(b_ref[...])
    ct = jnp.transpose(c_ref[...])
    mx = mx_ref[...]
    for j in range(NC_ROWS):
        cand = jnp.where(bt[:, j:j + 1] == giota,
                         ct[:, j:j + 1], -1)
        mx = jnp.maximum(mx, jnp.max(cand, axis=0, keepdims=True))
    mx_ref[...] = mx

    @pl.when(i == NPAD // (NC_ROWS * CHK) - 1)
    def _():
        o_ref[...] = (mx_ref[...] + 1).astype(jnp.float32)


def _mlp_body(e_ref, m_ref, w1_ref, b1_ref, w2_ref, b2_ref, o_ref):
    e = e_ref[...]
    h = jnp.dot(e, w1_ref[...], preferred_element_type=jnp.float32)
    h = h + b1_ref[...]
    h = jnp.where(h >= 0, h, 0.01 * h)
    val = jnp.dot(h, w2_ref[...],
                  preferred_element_type=jnp.float32) + b2_ref[0, 0]
    valm = val.reshape(-1, C_MAX)
    ciota = lax.broadcasted_iota(jnp.int32, (1, C_MAX), 1).astype(jnp.float32)
    msk = (ciota < m_ref[...]).astype(jnp.float32)
    v = jnp.sum(valm * msk, axis=1, keepdims=True)
    o_ref[...] = v


def kernel(node_embed, batch, component, W1, b1, W2, b2):
    bpad = jnp.full((NPAD - N,), B, jnp.int32)
    b2d = jnp.concatenate([batch, bpad]).reshape(NPAD // CHK, CHK)
    c2d = jnp.concatenate(
        [component, jnp.zeros((NPAD - N,), jnp.int32)]
    ).reshape(NPAD // CHK, CHK)
    numc = pl.pallas_call(
        _numc_body,
        grid=(NPAD // (NC_ROWS * CHK),),
        in_specs=[
            pl.BlockSpec((NC_ROWS, CHK), lambda i: (i, 0)),
            pl.BlockSpec((NC_ROWS, CHK), lambda i: (i, 0)),
        ],
        out_specs=pl.BlockSpec((1, B), lambda i: (0, 0)),
        out_shape=jax.ShapeDtypeStruct((1, B), jnp.float32),
        scratch_shapes=[pltpu.VMEM((1, B), jnp.int32)],
    )(b2d, c2d)
    numc = numc.reshape(B, 1)

    zeros = jnp.zeros((ZROWS, D // 2), jnp.float32)
    e = _sc_segment_sum(node_embed, batch, component, zeros)

    rows = 2048
    v = pl.pallas_call(
        _mlp_body,
        grid=(NBUCKET // rows,),
        in_specs=[
            pl.BlockSpec((rows, D), lambda i: (i, 0)),
            pl.BlockSpec((rows // C_MAX, 1), lambda i: (i, 0)),
            pl.BlockSpec((D, D), lambda i: (0, 0)),
            pl.BlockSpec((1, D), lambda i: (0, 0)),
            pl.BlockSpec((D, 1), lambda i: (0, 0)),
            pl.BlockSpec((1, 1), lambda i: (0, 0)),
        ],
        out_specs=pl.BlockSpec((rows // C_MAX, 1), lambda i: (i, 0)),
        out_shape=jax.ShapeDtypeStruct((B, 1), jnp.float32),
    )(e, numc, W1, b1.reshape(1, D), W2, b2.reshape(1, 1))
    return v


# speedup vs baseline: 1.0128x; 1.0128x over previous
import dataclasses
import functools

import jax
import jax.numpy as jnp
from jax import lax
from jax.experimental import pallas as pl
from jax.experimental.pallas import tpu as pltpu
from jax.experimental.pallas import tpu_sc as plsc

N = 50000
D = 256
B = 256
C_MAX = 32
CHUNK = 112
NCHUNKS = (N + CHUNK - 1) // CHUNK
NBUCKET = B * C_MAX
HALF = NBUCKET // 2
NSUB = 16
ZROWS = HALF // NSUB
KMAX = -(-NCHUNKS // NSUB)
NPAD = 49 * 1024


def _sc_body(node_hbm, batch_hbm, comp_hbm, zeros_hbm, e_out,
             nlo, nhi, bbufs, cbufs, ibufs, flags, acc_lo, acc_hi,
             semz, semh, semn0, semn1, sems0, sems1):
    cid = lax.axis_index("c")
    sid = lax.axis_index("s")
    base = cid * HALF
    glo = cid * (B // 2)
    semn = (semn0, semn1)
    sems = (sems0, sems1)

    def chunk_start(k):
        j = k * NSUB + sid
        return jnp.minimum(j * CHUNK, N - CHUNK)

    def header_dma(k):
        start = chunk_start(k)
        slot = lax.rem(k, 4) if isinstance(k, jax.Array) else k % 4
        return (pltpu.make_async_copy(batch_hbm.at[pl.ds(start, CHUNK)],
                                      bbufs.at[slot], semh),
                pltpu.make_async_copy(comp_hbm.at[pl.ds(start, CHUNK)],
                                      cbufs.at[slot], semh))

    zl = pltpu.make_async_copy(zeros_hbm, acc_lo.at[pl.ds(sid * ZROWS, ZROWS)],
                               semz)
    zh = pltpu.make_async_copy(zeros_hbm, acc_hi.at[pl.ds(sid * ZROWS, ZROWS)],
                               semz)
    zl.start()
    zh.start()
    for k in range(min(2, KMAX)):
        for h in header_dma(k):
            h.start()

    iota = lax.broadcasted_iota(jnp.int32, (16,), 0)
    dummy = HALF + sid
    HD = D // 2

    zl.wait()
    zh.wait()
    plsc.subcore_barrier()

    def node_dma(k, half):
        buf = lax.rem(k, 2)
        src = node_hbm.at[pl.ds(chunk_start(k), CHUNK), pl.ds(half * HD, HD)]
        return pltpu.make_async_copy(src, (nlo, nhi)[half].at[buf],
                                     semn[half])

    def scat_dma(k, half):
        buf = lax.rem(k, 2)
        return pltpu.make_async_copy(
            (nlo, nhi)[half].at[buf],
            (acc_lo, acc_hi)[half].at[ibufs.at[lax.rem(k, 4)]], sems[half])

    @pl.loop(0, KMAX + 2)
    def _(k):
        @pl.when((k >= 2) & (flags[jnp.maximum(k - 2, 0)] == 1))
        def _():
            scat_dma(k - 2, 0).wait()
            scat_dma(k - 2, 1).wait()

        @pl.when(k + 2 < KMAX)
        def _():
            for h in header_dma(k + 2):
                h.start()

        @pl.when(k < KMAX)
        def _():
            j = k * NSUB + sid
            start = chunk_start(k)
            slot = lax.rem(k, 4)
            for h in header_dma(k):
                h.wait()
            lo = jnp.min(bbufs[slot, pl.ds(0, 16)])
            hi = jnp.max(bbufs[slot, pl.ds(CHUNK - 16, 16)])
            rel = (hi >= glo) & (lo < glo + B // 2) & (j < NCHUNKS)
            flags[k] = rel.astype(jnp.int32)

            @pl.when(rel)
            def _():
                node_dma(k, 0).start()
                node_dma(k, 1).start()
                for g in range(CHUNK // 16):
                    b = bbufs[slot, pl.ds(g * 16, 16)]
                    c = cbufs[slot, pl.ds(g * 16, 16)]
                    sval = b * C_MAX + c - base
                    pos = start + g * 16 + iota
                    ok = (sval >= 0) & (sval < HALF) & (pos >= j * CHUNK)
                    ibufs[slot, pl.ds(g * 16, 16)] = jnp.where(ok, sval, dummy)

        @pl.when((k >= 1) & (k <= KMAX)
                 & (flags[jnp.maximum(k - 1, 0)] == 1))
        def _():
            node_dma(k - 1, 0).wait()
            node_dma(k - 1, 1).wait()
            scat_dma(k - 1, 0).start(add=True)
            scat_dma(k - 1, 1).start(add=True)

    plsc.subcore_barrier()
    row = sid * ZROWS
    pltpu.sync_copy(acc_lo.at[pl.ds(row, ZROWS)],
                    e_out.at[pl.ds(base + row, ZROWS), pl.ds(0, HD)])
    pltpu.sync_copy(acc_hi.at[pl.ds(row, ZROWS)],
                    e_out.at[pl.ds(base + row, ZROWS), pl.ds(HD, HD)])


_sc_compiler_params = pltpu.CompilerParams()
if "needs_layout_passes" in pltpu.CompilerParams.__dataclass_fields__:
    _sc_compiler_params = dataclasses.replace(
        _sc_compiler_params, needs_layout_passes=False)

_sc_segment_sum = functools.partial(
    pl.kernel,
    compiler_params=_sc_compiler_params,
    out_type=jax.ShapeDtypeStruct((NBUCKET, D), jnp.float32),
    mesh=plsc.VectorSubcoreMesh(core_axis_name="c", subcore_axis_name="s"),
    scratch_types=[
        pltpu.VMEM((2, CHUNK, D // 2), jnp.float32),
        pltpu.VMEM((2, CHUNK, D // 2), jnp.float32),
        pltpu.VMEM((4, CHUNK), jnp.int32),
        pltpu.VMEM((4, CHUNK), jnp.int32),
        pltpu.VMEM((4, CHUNK), jnp.int32),
        pltpu.SMEM((KMAX,), jnp.int32),
        pltpu.VMEM_SHARED((HALF + NSUB, D // 2), jnp.float32),
        pltpu.VMEM_SHARED((HALF + NSUB, D // 2), jnp.float32),
        pltpu.SemaphoreType.DMA,
        pltpu.SemaphoreType.DMA,
        pltpu.SemaphoreType.DMA,
        pltpu.SemaphoreType.DMA,
        pltpu.SemaphoreType.DMA,
        pltpu.SemaphoreType.DMA,
    ],
)(_sc_body)


NC_ROWS = 8
CHK = 128


def _numc_body(b_ref, c_ref, o_ref, mx_ref):
    i = pl.program_id(0)

    @pl.when(i == 0)
    def _():
        mx_ref[...] = jnp.full((1, B), -1, jnp.int32)

    giota = lax.broadcasted_iota(jnp.int32, (1, B), 1)
    bt = jnp.transpose(b_ref[...])
    ct = jnp.transpose(c_ref[...])
    mx = mx_ref[...]
    for j in range(NC_ROWS):
        cand = jnp.where(bt[:, j:j + 1] == giota,
                         ct[:, j:j + 1], -1)
        mx = jnp.maximum(mx, jnp.max(cand, axis=0, keepdims=True))
    mx_ref[...] = mx

    @pl.when(i == NPAD // (NC_ROWS * CHK) - 1)
    def _():
        o_ref[...] = (mx_ref[...] + 1).astype(jnp.float32)


def _mlp_body(e_ref, m_ref, w1_ref, b1_ref, w2_ref, b2_ref, o_ref):
    e = e_ref[...]
    h = jnp.dot(e, w1_ref[...], preferred_element_type=jnp.float32)
    h = h + b1_ref[...]
    h = jnp.where(h >= 0, h, 0.01 * h)
    val = jnp.dot(h, w2_ref[...],
                  preferred_element_type=jnp.float32) + b2_ref[0, 0]
    valm = val.reshape(-1, C_MAX)
    ciota = lax.broadcasted_iota(jnp.int32, (1, C_MAX), 1).astype(jnp.float32)
    msk = (ciota < m_ref[...]).astype(jnp.float32)
    v = jnp.sum(valm * msk, axis=1, keepdims=True)
    o_ref[...] = v


def kernel(node_embed, batch, component, W1, b1, W2, b2):
    bpad = jnp.full((NPAD - N,), B, jnp.int32)
    b2d = jnp.concatenate([batch, bpad]).reshape(NPAD // CHK, CHK)
    c2d = jnp.concatenate(
        [component, jnp.zeros((NPAD - N,), jnp.int32)]
    ).reshape(NPAD // CHK, CHK)
    numc = pl.pallas_call(
        _numc_body,
        grid=(NPAD // (NC_ROWS * CHK),),
        in_specs=[
            pl.BlockSpec((NC_ROWS, CHK), lambda i: (i, 0)),
            pl.BlockSpec((NC_ROWS, CHK), lambda i: (i, 0)),
        ],
        out_specs=pl.BlockSpec((1, B), lambda i: (0, 0)),
        out_shape=jax.ShapeDtypeStruct((1, B), jnp.float32),
        scratch_shapes=[pltpu.VMEM((1, B), jnp.int32)],
    )(b2d, c2d)
    numc = numc.reshape(B, 1)

    zeros = jnp.zeros((ZROWS, D // 2), jnp.float32)
    e = _sc_segment_sum(node_embed, batch, component, zeros)

    rows = 2048
    v = pl.pallas_call(
        _mlp_body,
        grid=(NBUCKET // rows,),
        in_specs=[
            pl.BlockSpec((rows, D), lambda i: (i, 0)),
            pl.BlockSpec((rows // C_MAX, 1), lambda i: (i, 0)),
            pl.BlockSpec((D, D), lambda i: (0, 0)),
            pl.BlockSpec((1, D), lambda i: (0, 0)),
            pl.BlockSpec((D, 1), lambda i: (0, 0)),
            pl.BlockSpec((1, 1), lambda i: (0, 0)),
        ],
        out_specs=pl.BlockSpec((rows // C_MAX, 1), lambda i: (i, 0)),
        out_shape=jax.ShapeDtypeStruct((B, 1), jnp.float32),
    )(e, numc, W1, b1.reshape(1, D), W2, b2.reshape(1, 1))
    return v
